# Initial kernel scaffold; baseline (speedup 1.0000x reference)
import jax
import jax.numpy as jnp
from jax.experimental import pallas as pl


def kernel(features, labels):
    raise NotImplementedError("write your pallas kernel here")


# speedup vs baseline: 5.3948x; 5.3948x over previous
import jax
import jax.numpy as jnp
from jax.experimental import pallas as pl
from jax.experimental.pallas import tpu as pltpu

_N_PIX = 320000
_D = 128
_C = 32
_B = 8000
_NB = _N_PIX // _B
_DELTA_VAR = 0.5
_DELTA_DIST = 1.5
_GAMMA = 0.001


def _pass_a(feat_ref, lab_ref, sums_ref, counts_ref):
    i = pl.program_id(0)
    f = feat_ref[...]
    lab = lab_ref[...]
    oh = (lab == jax.lax.broadcasted_iota(jnp.int32, (_B, _C), 1)).astype(
        jnp.float32)
    psum = jax.lax.dot_general(oh, f, (((0,), (0,)), ((), ())),
                               preferred_element_type=jnp.float32)
    pcnt = jax.lax.dot_general(oh, jnp.ones((_B, 1), jnp.float32),
                               (((0,), (0,)), ((), ())),
                               preferred_element_type=jnp.float32)

    @pl.when(i == 0)
    def _():
        sums_ref[...] = jnp.zeros_like(sums_ref)
        counts_ref[...] = jnp.zeros_like(counts_ref)

    sums_ref[...] += psum
    counts_ref[...] += pcnt


def _pass_b(feat_ref, lab_ref, sums_ref, counts_ref, out_ref,
            means_ref, acc_ref):
    i = pl.program_id(0)

    @pl.when(i == 0)
    def _():
        means_ref[...] = sums_ref[...] / counts_ref[...]
        acc_ref[...] = jnp.zeros_like(acc_ref)

    f = feat_ref[...]
    lab = lab_ref[...]
    means = means_ref[...]
    oh = (lab == jax.lax.broadcasted_iota(jnp.int32, (_B, _C), 1)).astype(
        jnp.float32)

    fs = jnp.sum(f * f, axis=1, keepdims=True)
    proj = jax.lax.dot_general(f, means, (((1,), (1,)), ((), ())),
                               preferred_element_type=jnp.float32)
    dot = jnp.sum(proj * oh, axis=1, keepdims=True)
    m2 = jnp.sum(means * means, axis=1, keepdims=True)
    m2g = jax.lax.dot_general(oh, m2, (((1,), (0,)), ((), ())),
                              preferred_element_type=jnp.float32)
    d2 = jnp.maximum(fs - 2.0 * dot + m2g, 0.0)
    hinge = jnp.maximum(jnp.sqrt(d2) - _DELTA_VAR, 0.0)
    h2 = hinge * hinge
    acc_ref[...] += jax.lax.dot_general(oh, h2, (((0,), (0,)), ((), ())),
                                        preferred_element_type=jnp.float32)

    @pl.when(i == _NB - 1)
    def _():
        counts = counts_ref[...]
        var_loss = jnp.sum(acc_ref[...] / counts) / _C

        gram = jax.lax.dot_general(means, means, (((1,), (1,)), ((), ())),
                                   preferred_element_type=jnp.float32)
        ri = jax.lax.broadcasted_iota(jnp.int32, (_C, _C), 0)
        ci = jax.lax.broadcasted_iota(jnp.int32, (_C, _C), 1)
        eye = ri == ci
        diag_c = jnp.sum(jnp.where(eye, gram, 0.0), axis=1, keepdims=True)
        diag_r = jnp.sum(jnp.where(eye, gram, 0.0), axis=0, keepdims=True)
        d2c = jnp.maximum(diag_c + diag_r - 2.0 * gram, 0.0)
        dmat = jnp.sqrt(jnp.where(eye, 1.0, d2c))
        hd = jnp.maximum(2.0 * _DELTA_DIST - dmat, 0.0) ** 2
        hd = jnp.where(eye, 0.0, hd)
        dist_loss = jnp.sum(hd) / float((_C - 1) * _C)

        reg_loss = jnp.sum(jnp.sqrt(jnp.maximum(m2, 0.0))) / _C

        loss = var_loss + dist_loss + _GAMMA * reg_loss

        li = jax.lax.broadcasted_iota(jnp.int32, (8, 128), 1)
        si = jax.lax.broadcasted_iota(jnp.int32, (8, 128), 0)
        vals = jnp.where(li == 0, loss,
                         jnp.where(li == 1, var_loss,
                                   jnp.where(li == 2, dist_loss,
                                             jnp.where(li == 3, reg_loss,
                                                       0.0))))
        out_ref[...] = jnp.where(si == 0, vals, 0.0)


def kernel(features, labels):
    labels_2d = labels.reshape(_N_PIX, 1)

    sums, counts = pl.pallas_call(
        _pass_a,
        grid=(_NB,),
        in_specs=[
            pl.BlockSpec((_B, _D), lambda i: (i, 0)),
            pl.BlockSpec((_B, 1), lambda i: (i, 0)),
        ],
        out_specs=[
            pl.BlockSpec((_C, _D), lambda i: (0, 0)),
            pl.BlockSpec((_C, 1), lambda i: (0, 0)),
        ],
        out_shape=[
            jax.ShapeDtypeStruct((_C, _D), jnp.float32),
            jax.ShapeDtypeStruct((_C, 1), jnp.float32),
        ],
    )(features, labels_2d)

    out = pl.pallas_call(
        _pass_b,
        grid=(_NB,),
        in_specs=[
            pl.BlockSpec((_B, _D), lambda i: (i, 0)),
            pl.BlockSpec((_B, 1), lambda i: (i, 0)),
            pl.BlockSpec((_C, _D), lambda i: (0, 0)),
            pl.BlockSpec((_C, 1), lambda i: (0, 0)),
        ],
        out_specs=pl.BlockSpec((8, 128), lambda i: (0, 0)),
        out_shape=jax.ShapeDtypeStruct((8, 128), jnp.float32),
        scratch_shapes=[
            pltpu.VMEM((_C, _D), jnp.float32),
            pltpu.VMEM((_C, 1), jnp.float32),
        ],
    )(features, labels_2d, sums, counts)

    return out[0, :4]


# speedup vs baseline: 7.2721x; 1.3480x over previous
import jax
import jax.numpy as jnp
from jax.experimental import pallas as pl
from jax.experimental.pallas import tpu as pltpu

_N_PIX = 320000
_D = 128
_C = 32
_B = 8000
_NB = _N_PIX // _B
_DELTA_VAR = 0.5
_DELTA_DIST = 1.5
_GAMMA = 0.001


def _onehot_bc(lab_col):
    return (lab_col == jax.lax.broadcasted_iota(jnp.int32, (_B, _C), 1)
            ).astype(jnp.bfloat16)


def _onehot_cb(lab_row):
    return (lab_row == jax.lax.broadcasted_iota(jnp.int32, (_C, _B), 0)
            ).astype(jnp.bfloat16)


def _pass_a(feat_ref, labr_ref, sums_ref, counts_ref):
    i = pl.program_id(0)
    f = feat_ref[...].astype(jnp.bfloat16)
    oht = _onehot_cb(labr_ref[...].reshape(1, _B))
    psum = jax.lax.dot_general(oht, f, (((1,), (0,)), ((), ())),
                               preferred_element_type=jnp.float32)
    pcnt = jax.lax.dot_general(oht, jnp.ones((_B, 1), jnp.bfloat16),
                               (((1,), (0,)), ((), ())),
                               preferred_element_type=jnp.float32)

    @pl.when(i == 0)
    def _():
        sums_ref[...] = jnp.zeros_like(sums_ref)
        counts_ref[...] = jnp.zeros_like(counts_ref)

    sums_ref[...] += psum
    counts_ref[...] += pcnt


def _pass_b(feat_ref, labc_ref, labr_ref, sums_ref, counts_ref, out_ref,
            means_ref, acc_ref):
    i = pl.program_id(0)

    @pl.when(i == 0)
    def _():
        means_ref[...] = sums_ref[...] / counts_ref[...]
        acc_ref[...] = jnp.zeros_like(acc_ref)

    f = feat_ref[...]
    means = means_ref[...]
    oh = _onehot_bc(labc_ref[...])
    oht = _onehot_cb(labr_ref[...].reshape(1, _B))

    mu_g = jax.lax.dot_general(oh, means.astype(jnp.bfloat16),
                               (((1,), (0,)), ((), ())),
                               preferred_element_type=jnp.float32)
    diff = f - mu_g
    d2 = jnp.sum(diff * diff, axis=1, keepdims=True)
    hinge = jnp.maximum(jnp.sqrt(d2) - _DELTA_VAR, 0.0)
    h2 = (hinge * hinge).astype(jnp.float32)
    acc_ref[...] += jax.lax.dot_general(oht.astype(jnp.float32), h2,
                                        (((1,), (0,)), ((), ())),
                                        preferred_element_type=jnp.float32)

    @pl.when(i == _NB - 1)
    def _():
        counts = counts_ref[...]
        var_loss = jnp.sum(acc_ref[...] / counts) / _C

        m2 = jnp.sum(means * means, axis=1, keepdims=True)
        gram = jax.lax.dot_general(means, means, (((1,), (1,)), ((), ())),
                                   preferred_element_type=jnp.float32)
        ri = jax.lax.broadcasted_iota(jnp.int32, (_C, _C), 0)
        ci = jax.lax.broadcasted_iota(jnp.int32, (_C, _C), 1)
        eye = ri == ci
        diag_c = jnp.sum(jnp.where(eye, gram, 0.0), axis=1, keepdims=True)
        diag_r = jnp.sum(jnp.where(eye, gram, 0.0), axis=0, keepdims=True)
        d2c = jnp.maximum(diag_c + diag_r - 2.0 * gram, 0.0)
        dmat = jnp.sqrt(jnp.where(eye, 1.0, d2c))
        hd = jnp.maximum(2.0 * _DELTA_DIST - dmat, 0.0) ** 2
        hd = jnp.where(eye, 0.0, hd)
        dist_loss = jnp.sum(hd) / float((_C - 1) * _C)

        reg_loss = jnp.sum(jnp.sqrt(jnp.maximum(m2, 0.0))) / _C

        loss = var_loss + dist_loss + _GAMMA * reg_loss

        li = jax.lax.broadcasted_iota(jnp.int32, (8, 128), 1)
        si = jax.lax.broadcasted_iota(jnp.int32, (8, 128), 0)
        vals = jnp.where(li == 0, loss,
                         jnp.where(li == 1, var_loss,
                                   jnp.where(li == 2, dist_loss,
                                             jnp.where(li == 3, reg_loss,
                                                       0.0))))
        out_ref[...] = jnp.where(si == 0, vals, 0.0)


def kernel(features, labels):
    labels_col = labels.reshape(_N_PIX, 1)
    labels_row = labels.reshape(_NB, 1, _B)

    sums, counts = pl.pallas_call(
        _pass_a,
        grid=(_NB,),
        in_specs=[
            pl.BlockSpec((_B, _D), lambda i: (i, 0)),
            pl.BlockSpec((1, 1, _B), lambda i: (i, 0, 0)),
        ],
        out_specs=[
            pl.BlockSpec((_C, _D), lambda i: (0, 0)),
            pl.BlockSpec((_C, 1), lambda i: (0, 0)),
        ],
        out_shape=[
            jax.ShapeDtypeStruct((_C, _D), jnp.float32),
            jax.ShapeDtypeStruct((_C, 1), jnp.float32),
        ],
    )(features, labels_row)

    out = pl.pallas_call(
        _pass_b,
        grid=(_NB,),
        in_specs=[
            pl.BlockSpec((_B, _D), lambda i: (i, 0)),
            pl.BlockSpec((_B, 1), lambda i: (i, 0)),
            pl.BlockSpec((1, 1, _B), lambda i: (i, 0, 0)),
            pl.BlockSpec((_C, _D), lambda i: (0, 0)),
            pl.BlockSpec((_C, 1), lambda i: (0, 0)),
        ],
        out_specs=pl.BlockSpec((8, 128), lambda i: (0, 0)),
        out_shape=jax.ShapeDtypeStruct((8, 128), jnp.float32),
        scratch_shapes=[
            pltpu.VMEM((_C, _D), jnp.float32),
            pltpu.VMEM((_C, 1), jnp.float32),
        ],
    )(features, labels_col, labels_row, sums, counts)

    return out[0, :4]


# speedup vs baseline: 10.8818x; 1.4964x over previous
import jax
import jax.numpy as jnp
from jax.experimental import pallas as pl
from jax.experimental.pallas import tpu as pltpu

_N_PIX = 320000
_D = 128
_C = 32
_B = 16000
_NB = _N_PIX // _B
_DELTA_VAR = 0.5
_DELTA_DIST = 1.5
_GAMMA = 0.001


def _onehot_cb(lab_row, dtype):
    return (lab_row == jax.lax.broadcasted_iota(jnp.int32, (_C, _B), 0)
            ).astype(dtype)


def _pass_a(feat_ref, labr_ref, sums_ref, counts_ref, fs_ref):
    i = pl.program_id(0)
    f = feat_ref[...]
    fb = f.astype(jnp.bfloat16)
    oht = _onehot_cb(labr_ref[...].reshape(1, _B), jnp.bfloat16)
    psum = jax.lax.dot_general(oht, fb, (((1,), (0,)), ((), ())),
                               preferred_element_type=jnp.float32)
    pcnt = jax.lax.dot_general(oht, jnp.ones((_B, 1), jnp.bfloat16),
                               (((1,), (0,)), ((), ())),
                               preferred_element_type=jnp.float32)
    fs_ref[...] = jnp.sum(f * f, axis=1, keepdims=True)

    @pl.when(i == 0)
    def _():
        sums_ref[...] = jnp.zeros_like(sums_ref)
        counts_ref[...] = jnp.zeros_like(counts_ref)

    sums_ref[...] += psum
    counts_ref[...] += pcnt


def _pass_b(feat_ref, labr_ref, fsr_ref, sums_ref, counts_ref, out_ref,
            means_ref, acc_ref):
    i = pl.program_id(0)

    @pl.when(i == 0)
    def _():
        means_ref[...] = sums_ref[...] / counts_ref[...]
        acc_ref[...] = jnp.zeros_like(acc_ref)

    means = means_ref[...]
    mb = means.astype(jnp.bfloat16)
    fb = feat_ref[...].astype(jnp.bfloat16)
    oht = _onehot_cb(labr_ref[...].reshape(1, _B), jnp.bfloat16)
    fs_row = fsr_ref[...].reshape(1, _B)

    projt = jax.lax.dot_general(mb, fb, (((1,), (1,)), ((), ())),
                                preferred_element_type=jnp.float32)
    m2 = jnp.sum(means * means, axis=1, keepdims=True)
    sel = oht.astype(jnp.float32) * (m2 - 2.0 * projt)
    corr = jnp.sum(sel, axis=0, keepdims=True)
    d2 = jnp.maximum(fs_row + corr, 0.0)
    s = d2 * jax.lax.rsqrt(jnp.maximum(d2, 1e-12))
    hinge = jnp.maximum(s - _DELTA_VAR, 0.0)
    h2 = (hinge * hinge).astype(jnp.bfloat16)
    contrib = oht * h2
    acc_ref[...] += jax.lax.dot_general(contrib, jnp.ones((_B, 1), jnp.bfloat16),
                                        (((1,), (0,)), ((), ())),
                                        preferred_element_type=jnp.float32)

    @pl.when(i == _NB - 1)
    def _():
        counts = counts_ref[...]
        var_loss = jnp.sum(acc_ref[...] / counts) / _C

        gram = jax.lax.dot_general(means, means, (((1,), (1,)), ((), ())),
                                   preferred_element_type=jnp.float32)
        ri = jax.lax.broadcasted_iota(jnp.int32, (_C, _C), 0)
        ci = jax.lax.broadcasted_iota(jnp.int32, (_C, _C), 1)
        eye = ri == ci
        diag_c = jnp.sum(jnp.where(eye, gram, 0.0), axis=1, keepdims=True)
        diag_r = jnp.sum(jnp.where(eye, gram, 0.0), axis=0, keepdims=True)
        d2c = jnp.maximum(diag_c + diag_r - 2.0 * gram, 0.0)
        dmat = jnp.sqrt(jnp.where(eye, 1.0, d2c))
        hd = jnp.maximum(2.0 * _DELTA_DIST - dmat, 0.0) ** 2
        hd = jnp.where(eye, 0.0, hd)
        dist_loss = jnp.sum(hd) / float((_C - 1) * _C)

        reg_loss = jnp.sum(jnp.sqrt(jnp.maximum(m2, 0.0))) / _C

        loss = var_loss + dist_loss + _GAMMA * reg_loss

        li = jax.lax.broadcasted_iota(jnp.int32, (8, 128), 1)
        si = jax.lax.broadcasted_iota(jnp.int32, (8, 128), 0)
        vals = jnp.where(li == 0, loss,
                         jnp.where(li == 1, var_loss,
                                   jnp.where(li == 2, dist_loss,
                                             jnp.where(li == 3, reg_loss,
                                                       0.0))))
        out_ref[...] = jnp.where(si == 0, vals, 0.0)


def kernel(features, labels):
    labels_row = labels.reshape(_NB, 1, _B)

    sums, counts, fs = pl.pallas_call(
        _pass_a,
        grid=(_NB,),
        in_specs=[
            pl.BlockSpec((_B, _D), lambda i: (i, 0)),
            pl.BlockSpec((1, 1, _B), lambda i: (i, 0, 0)),
        ],
        out_specs=[
            pl.BlockSpec((_C, _D), lambda i: (0, 0)),
            pl.BlockSpec((_C, 1), lambda i: (0, 0)),
            pl.BlockSpec((_B, 1), lambda i: (i, 0)),
        ],
        out_shape=[
            jax.ShapeDtypeStruct((_C, _D), jnp.float32),
            jax.ShapeDtypeStruct((_C, 1), jnp.float32),
            jax.ShapeDtypeStruct((_N_PIX, 1), jnp.float32),
        ],
    )(features, labels_row)

    fs_row = fs.reshape(_NB, 1, _B)

    out = pl.pallas_call(
        _pass_b,
        grid=(_NB,),
        in_specs=[
            pl.BlockSpec((_B, _D), lambda i: (i, 0)),
            pl.BlockSpec((1, 1, _B), lambda i: (i, 0, 0)),
            pl.BlockSpec((1, 1, _B), lambda i: (i, 0, 0)),
            pl.BlockSpec((_C, _D), lambda i: (0, 0)),
            pl.BlockSpec((_C, 1), lambda i: (0, 0)),
        ],
        out_specs=pl.BlockSpec((8, 128), lambda i: (0, 0)),
        out_shape=jax.ShapeDtypeStruct((8, 128), jnp.float32),
        scratch_shapes=[
            pltpu.VMEM((_C, _D), jnp.float32),
            pltpu.VMEM((_C, 1), jnp.float32),
        ],
    )(features, labels_row, fs_row, sums, counts)

    return out[0, :4]


# speedup vs baseline: 10.9466x; 1.0060x over previous
import jax
import jax.numpy as jnp
from jax.experimental import pallas as pl
from jax.experimental.pallas import tpu as pltpu

_N_PIX = 320000
_D = 128
_C = 32
_B = 20000
_NB = _N_PIX // _B
_DELTA_VAR = 0.5
_DELTA_DIST = 1.5
_GAMMA = 0.001


def _onehot_cb(lab_row, dtype):
    return (lab_row == jax.lax.broadcasted_iota(jnp.int32, (_C, _B), 0)
            ).astype(dtype)


def _pass_a(feat_ref, labr_ref, sums_ref, counts_ref, fs_ref):
    i = pl.program_id(0)
    f = feat_ref[...]
    fb = f.astype(jnp.bfloat16)
    oht = _onehot_cb(labr_ref[...].reshape(1, _B), jnp.bfloat16)
    psum = jax.lax.dot_general(oht, fb, (((1,), (0,)), ((), ())),
                               preferred_element_type=jnp.float32)
    pcnt = jax.lax.dot_general(oht, jnp.ones((_B, 1), jnp.bfloat16),
                               (((1,), (0,)), ((), ())),
                               preferred_element_type=jnp.float32)
    fs_ref[...] = jnp.sum(f * f, axis=1, keepdims=True)

    @pl.when(i == 0)
    def _():
        sums_ref[...] = jnp.zeros_like(sums_ref)
        counts_ref[...] = jnp.zeros_like(counts_ref)

    sums_ref[...] += psum
    counts_ref[...] += pcnt


def _pass_b(feat_ref, labr_ref, fsr_ref, sums_ref, counts_ref, out_ref,
            means_ref, acc_ref):
    i = pl.program_id(0)

    @pl.when(i == 0)
    def _():
        means_ref[...] = sums_ref[...] / counts_ref[...]
        acc_ref[...] = jnp.zeros_like(acc_ref)

    means = means_ref[...]
    mb = means.astype(jnp.bfloat16)
    fb = feat_ref[...].astype(jnp.bfloat16)
    oht = _onehot_cb(labr_ref[...].reshape(1, _B), jnp.bfloat16)
    fs_row = fsr_ref[...].reshape(1, _B)

    projt = jax.lax.dot_general(mb, fb, (((1,), (1,)), ((), ())),
                                preferred_element_type=jnp.float32)
    m2 = jnp.sum(means * means, axis=1, keepdims=True)
    sel = oht.astype(jnp.float32) * (m2 - 2.0 * projt)
    corr = jnp.sum(sel, axis=0, keepdims=True)
    d2 = jnp.maximum(fs_row + corr, 0.0)
    s = d2 * jax.lax.rsqrt(jnp.maximum(d2, 1e-12))
    hinge = jnp.maximum(s - _DELTA_VAR, 0.0)
    h2 = (hinge * hinge).astype(jnp.bfloat16)
    contrib = oht * h2
    acc_ref[...] += jax.lax.dot_general(contrib, jnp.ones((_B, 1), jnp.bfloat16),
                                        (((1,), (0,)), ((), ())),
                                        preferred_element_type=jnp.float32)

    @pl.when(i == _NB - 1)
    def _():
        counts = counts_ref[...]
        var_loss = jnp.sum(acc_ref[...] / counts) / _C

        gram = jax.lax.dot_general(means, means, (((1,), (1,)), ((), ())),
                                   preferred_element_type=jnp.float32)
        ri = jax.lax.broadcasted_iota(jnp.int32, (_C, _C), 0)
        ci = jax.lax.broadcasted_iota(jnp.int32, (_C, _C), 1)
        eye = ri == ci
        diag_c = jnp.sum(jnp.where(eye, gram, 0.0), axis=1, keepdims=True)
        diag_r = jnp.sum(jnp.where(eye, gram, 0.0), axis=0, keepdims=True)
        d2c = jnp.maximum(diag_c + diag_r - 2.0 * gram, 0.0)
        dmat = jnp.sqrt(jnp.where(eye, 1.0, d2c))
        hd = jnp.maximum(2.0 * _DELTA_DIST - dmat, 0.0) ** 2
        hd = jnp.where(eye, 0.0, hd)
        dist_loss = jnp.sum(hd) / float((_C - 1) * _C)

        reg_loss = jnp.sum(jnp.sqrt(jnp.maximum(m2, 0.0))) / _C

        loss = var_loss + dist_loss + _GAMMA * reg_loss

        li = jax.lax.broadcasted_iota(jnp.int32, (8, 128), 1)
        si = jax.lax.broadcasted_iota(jnp.int32, (8, 128), 0)
        vals = jnp.where(li == 0, loss,
                         jnp.where(li == 1, var_loss,
                                   jnp.where(li == 2, dist_loss,
                                             jnp.where(li == 3, reg_loss,
                                                       0.0))))
        out_ref[...] = jnp.where(si == 0, vals, 0.0)


def kernel(features, labels):
    labels_row = labels.reshape(_NB, 1, _B)

    sums, counts, fs = pl.pallas_call(
        _pass_a,
        grid=(_NB,),
        in_specs=[
            pl.BlockSpec((_B, _D), lambda i: (i, 0)),
            pl.BlockSpec((1, 1, _B), lambda i: (i, 0, 0)),
        ],
        out_specs=[
            pl.BlockSpec((_C, _D), lambda i: (0, 0)),
            pl.BlockSpec((_C, 1), lambda i: (0, 0)),
            pl.BlockSpec((_B, 1), lambda i: (i, 0)),
        ],
        out_shape=[
            jax.ShapeDtypeStruct((_C, _D), jnp.float32),
            jax.ShapeDtypeStruct((_C, 1), jnp.float32),
            jax.ShapeDtypeStruct((_N_PIX, 1), jnp.float32),
        ],
    )(features, labels_row)

    fs_row = fs.reshape(_NB, 1, _B)

    out = pl.pallas_call(
        _pass_b,
        grid=(_NB,),
        in_specs=[
            pl.BlockSpec((_B, _D), lambda i: (i, 0)),
            pl.BlockSpec((1, 1, _B), lambda i: (i, 0, 0)),
            pl.BlockSpec((1, 1, _B), lambda i: (i, 0, 0)),
            pl.BlockSpec((_C, _D), lambda i: (0, 0)),
            pl.BlockSpec((_C, 1), lambda i: (0, 0)),
        ],
        out_specs=pl.BlockSpec((8, 128), lambda i: (0, 0)),
        out_shape=jax.ShapeDtypeStruct((8, 128), jnp.float32),
        scratch_shapes=[
            pltpu.VMEM((_C, _D), jnp.float32),
            pltpu.VMEM((_C, 1), jnp.float32),
        ],
    )(features, labels_row, fs_row, sums, counts)

    return out[0, :4]


# speedup vs baseline: 19.8040x; 1.8091x over previous
import jax
import jax.numpy as jnp
from jax.experimental import pallas as pl
from jax.experimental.pallas import tpu as pltpu

_N_PIX = 320000
_D = 128
_C = 32
_B = 20000
_NB = _N_PIX // _B
_DELTA_VAR = 0.5
_DELTA_DIST = 1.5
_GAMMA = 0.001


def _onehot_cb(lab_row, dtype):
    return (lab_row == jax.lax.broadcasted_iota(jnp.int32, (_C, _B), 0)
            ).astype(dtype)


def _pass_a(feat_ref, labr_ref, sums_ref, counts_ref, fs_ref):
    i = pl.program_id(0)
    f = feat_ref[...]
    fb = f.astype(jnp.bfloat16)
    oht = _onehot_cb(labr_ref[...].reshape(1, _B), jnp.bfloat16)
    psum = jax.lax.dot_general(oht, fb, (((1,), (0,)), ((), ())),
                               preferred_element_type=jnp.float32)
    pcnt = jax.lax.dot_general(oht, jnp.ones((_B, 1), jnp.bfloat16),
                               (((1,), (0,)), ((), ())),
                               preferred_element_type=jnp.float32)
    f2b = fb * fb
    fs_row = jax.lax.dot_general(jnp.ones((1, _D), jnp.bfloat16), f2b,
                                 (((1,), (1,)), ((), ())),
                                 preferred_element_type=jnp.float32)
    fs_ref[...] = fs_row.reshape(1, 1, _B)

    @pl.when(i == 0)
    def _():
        sums_ref[...] = jnp.zeros_like(sums_ref)
        counts_ref[...] = jnp.zeros_like(counts_ref)

    sums_ref[...] += psum
    counts_ref[...] += pcnt


def _pass_b(feat_ref, labr_ref, fsr_ref, sums_ref, counts_ref, out_ref,
            means_ref, acc_ref):
    i = pl.program_id(0)

    @pl.when(i == 0)
    def _():
        means_ref[...] = sums_ref[...] / counts_ref[...]
        acc_ref[...] = jnp.zeros_like(acc_ref)

    means = means_ref[...]
    mb = means.astype(jnp.bfloat16)
    fb = feat_ref[...].astype(jnp.bfloat16)
    oht = _onehot_cb(labr_ref[...].reshape(1, _B), jnp.bfloat16)
    fs_row = fsr_ref[...].reshape(1, _B)

    projt = jax.lax.dot_general(mb, fb, (((1,), (1,)), ((), ())),
                                preferred_element_type=jnp.float32)
    m2 = jnp.sum(means * means, axis=1, keepdims=True)
    sel = oht.astype(jnp.float32) * (m2 - 2.0 * projt)
    corr = jnp.sum(sel, axis=0, keepdims=True)
    d2 = jnp.maximum(fs_row + corr, 0.0)
    s = d2 * jax.lax.rsqrt(jnp.maximum(d2, 1e-12))
    hinge = jnp.maximum(s - _DELTA_VAR, 0.0)
    h2 = (hinge * hinge).astype(jnp.bfloat16)
    contrib = oht * h2
    acc_ref[...] += jax.lax.dot_general(contrib,
                                        jnp.ones((_B, 1), jnp.bfloat16),
                                        (((1,), (0,)), ((), ())),
                                        preferred_element_type=jnp.float32)

    @pl.when(i == _NB - 1)
    def _():
        counts = counts_ref[...]
        var_loss = jnp.sum(acc_ref[...] / counts) / _C

        gram = jax.lax.dot_general(means, means, (((1,), (1,)), ((), ())),
                                   preferred_element_type=jnp.float32)
        ri = jax.lax.broadcasted_iota(jnp.int32, (_C, _C), 0)
        ci = jax.lax.broadcasted_iota(jnp.int32, (_C, _C), 1)
        eye = ri == ci
        diag_c = jnp.sum(jnp.where(eye, gram, 0.0), axis=1, keepdims=True)
        diag_r = jnp.sum(jnp.where(eye, gram, 0.0), axis=0, keepdims=True)
        d2c = jnp.maximum(diag_c + diag_r - 2.0 * gram, 0.0)
        dmat = jnp.sqrt(jnp.where(eye, 1.0, d2c))
        hd = jnp.maximum(2.0 * _DELTA_DIST - dmat, 0.0) ** 2
        hd = jnp.where(eye, 0.0, hd)
        dist_loss = jnp.sum(hd) / float((_C - 1) * _C)

        reg_loss = jnp.sum(jnp.sqrt(jnp.maximum(m2, 0.0))) / _C

        loss = var_loss + dist_loss + _GAMMA * reg_loss

        li = jax.lax.broadcasted_iota(jnp.int32, (8, 128), 1)
        si = jax.lax.broadcasted_iota(jnp.int32, (8, 128), 0)
        vals = jnp.where(li == 0, loss,
                         jnp.where(li == 1, var_loss,
                                   jnp.where(li == 2, dist_loss,
                                             jnp.where(li == 3, reg_loss,
                                                       0.0))))
        out_ref[...] = jnp.where(si == 0, vals, 0.0)


def kernel(features, labels):
    labels_row = labels.reshape(_NB, 1, _B)

    sums, counts, fs_rows = pl.pallas_call(
        _pass_a,
        grid=(_NB,),
        in_specs=[
            pl.BlockSpec((_B, _D), lambda i: (i, 0)),
            pl.BlockSpec((1, 1, _B), lambda i: (i, 0, 0)),
        ],
        out_specs=[
            pl.BlockSpec((_C, _D), lambda i: (0, 0)),
            pl.BlockSpec((_C, 1), lambda i: (0, 0)),
            pl.BlockSpec((1, 1, _B), lambda i: (i, 0, 0)),
        ],
        out_shape=[
            jax.ShapeDtypeStruct((_C, _D), jnp.float32),
            jax.ShapeDtypeStruct((_C, 1), jnp.float32),
            jax.ShapeDtypeStruct((_NB, 1, _B), jnp.float32),
        ],
    )(features, labels_row)

    out = pl.pallas_call(
        _pass_b,
        grid=(_NB,),
        in_specs=[
            pl.BlockSpec((_B, _D), lambda i: (i, 0)),
            pl.BlockSpec((1, 1, _B), lambda i: (i, 0, 0)),
            pl.BlockSpec((1, 1, _B), lambda i: (i, 0, 0)),
            pl.BlockSpec((_C, _D), lambda i: (0, 0)),
            pl.BlockSpec((_C, 1), lambda i: (0, 0)),
        ],
        out_specs=pl.BlockSpec((8, 128), lambda i: (0, 0)),
        out_shape=jax.ShapeDtypeStruct((8, 128), jnp.float32),
        scratch_shapes=[
            pltpu.VMEM((_C, _D), jnp.float32),
            pltpu.VMEM((_C, 1), jnp.float32),
        ],
    )(features, labels_row, fs_rows, sums, counts)

    return out[0, :4]


# speedup vs baseline: 20.5899x; 1.0397x over previous
import jax
import jax.numpy as jnp
from jax.experimental import pallas as pl
from jax.experimental.pallas import tpu as pltpu

_N_PIX = 320000
_D = 128
_C = 32
_B = 32000
_NB = _N_PIX // _B
_DELTA_VAR = 0.5
_DELTA_DIST = 1.5
_GAMMA = 0.001


def _onehot_cb(lab_row, dtype):
    return (lab_row == jax.lax.broadcasted_iota(jnp.int32, (_C, _B), 0)
            ).astype(dtype)


def _pass_a(feat_ref, labr_ref, sums_ref, counts_ref, fs_ref):
    i = pl.program_id(0)
    f = feat_ref[...]
    fb = f.astype(jnp.bfloat16)
    oht = _onehot_cb(labr_ref[...].reshape(1, _B), jnp.bfloat16)
    psum = jax.lax.dot_general(oht, fb, (((1,), (0,)), ((), ())),
                               preferred_element_type=jnp.float32)
    pcnt = jax.lax.dot_general(oht, jnp.ones((_B, 1), jnp.bfloat16),
                               (((1,), (0,)), ((), ())),
                               preferred_element_type=jnp.float32)
    f2b = fb * fb
    fs_row = jax.lax.dot_general(jnp.ones((1, _D), jnp.bfloat16), f2b,
                                 (((1,), (1,)), ((), ())),
                                 preferred_element_type=jnp.float32)
    fs_ref[...] = fs_row.reshape(1, 1, _B)

    @pl.when(i == 0)
    def _():
        sums_ref[...] = jnp.zeros_like(sums_ref)
        counts_ref[...] = jnp.zeros_like(counts_ref)

    sums_ref[...] += psum
    counts_ref[...] += pcnt


def _pass_b(feat_ref, labr_ref, fsr_ref, sums_ref, counts_ref, out_ref,
            means_ref, acc_ref):
    i = pl.program_id(0)

    @pl.when(i == 0)
    def _():
        means_ref[...] = sums_ref[...] / counts_ref[...]
        acc_ref[...] = jnp.zeros_like(acc_ref)

    means = means_ref[...]
    mb = means.astype(jnp.bfloat16)
    fb = feat_ref[...].astype(jnp.bfloat16)
    oht = _onehot_cb(labr_ref[...].reshape(1, _B), jnp.bfloat16)
    fs_row = fsr_ref[...].reshape(1, _B)

    projt = jax.lax.dot_general(mb, fb, (((1,), (1,)), ((), ())),
                                preferred_element_type=jnp.float32)
    m2 = jnp.sum(means * means, axis=1, keepdims=True)
    sel = oht.astype(jnp.float32) * (m2 - 2.0 * projt)
    corr = jnp.sum(sel, axis=0, keepdims=True)
    d2 = jnp.maximum(fs_row + corr, 0.0)
    s = d2 * jax.lax.rsqrt(jnp.maximum(d2, 1e-12))
    hinge = jnp.maximum(s - _DELTA_VAR, 0.0)
    h2 = (hinge * hinge).astype(jnp.bfloat16)
    contrib = oht * h2
    acc_ref[...] += jax.lax.dot_general(contrib,
                                        jnp.ones((_B, 1), jnp.bfloat16),
                                        (((1,), (0,)), ((), ())),
                                        preferred_element_type=jnp.float32)

    @pl.when(i == _NB - 1)
    def _():
        counts = counts_ref[...]
        var_loss = jnp.sum(acc_ref[...] / counts) / _C

        gram = jax.lax.dot_general(means, means, (((1,), (1,)), ((), ())),
                                   preferred_element_type=jnp.float32)
        ri = jax.lax.broadcasted_iota(jnp.int32, (_C, _C), 0)
        ci = jax.lax.broadcasted_iota(jnp.int32, (_C, _C), 1)
        eye = ri == ci
        diag_c = jnp.sum(jnp.where(eye, gram, 0.0), axis=1, keepdims=True)
        diag_r = jnp.sum(jnp.where(eye, gram, 0.0), axis=0, keepdims=True)
        d2c = jnp.maximum(diag_c + diag_r - 2.0 * gram, 0.0)
        dmat = jnp.sqrt(jnp.where(eye, 1.0, d2c))
        hd = jnp.maximum(2.0 * _DELTA_DIST - dmat, 0.0) ** 2
        hd = jnp.where(eye, 0.0, hd)
        dist_loss = jnp.sum(hd) / float((_C - 1) * _C)

        reg_loss = jnp.sum(jnp.sqrt(jnp.maximum(m2, 0.0))) / _C

        loss = var_loss + dist_loss + _GAMMA * reg_loss

        li = jax.lax.broadcasted_iota(jnp.int32, (8, 128), 1)
        si = jax.lax.broadcasted_iota(jnp.int32, (8, 128), 0)
        vals = jnp.where(li == 0, loss,
                         jnp.where(li == 1, var_loss,
                                   jnp.where(li == 2, dist_loss,
                                             jnp.where(li == 3, reg_loss,
                                                       0.0))))
        out_ref[...] = jnp.where(si == 0, vals, 0.0)


def kernel(features, labels):
    labels_row = labels.reshape(_NB, 1, _B)

    sums, counts, fs_rows = pl.pallas_call(
        _pass_a,
        grid=(_NB,),
        in_specs=[
            pl.BlockSpec((_B, _D), lambda i: (i, 0)),
            pl.BlockSpec((1, 1, _B), lambda i: (i, 0, 0)),
        ],
        out_specs=[
            pl.BlockSpec((_C, _D), lambda i: (0, 0)),
            pl.BlockSpec((_C, 1), lambda i: (0, 0)),
            pl.BlockSpec((1, 1, _B), lambda i: (i, 0, 0)),
        ],
        out_shape=[
            jax.ShapeDtypeStruct((_C, _D), jnp.float32),
            jax.ShapeDtypeStruct((_C, 1), jnp.float32),
            jax.ShapeDtypeStruct((_NB, 1, _B), jnp.float32),
        ],
    )(features, labels_row)

    out = pl.pallas_call(
        _pass_b,
        grid=(_NB,),
        in_specs=[
            pl.BlockSpec((_B, _D), lambda i: (i, 0)),
            pl.BlockSpec((1, 1, _B), lambda i: (i, 0, 0)),
            pl.BlockSpec((1, 1, _B), lambda i: (i, 0, 0)),
            pl.BlockSpec((_C, _D), lambda i: (0, 0)),
            pl.BlockSpec((_C, 1), lambda i: (0, 0)),
        ],
        out_specs=pl.BlockSpec((8, 128), lambda i: (0, 0)),
        out_shape=jax.ShapeDtypeStruct((8, 128), jnp.float32),
        scratch_shapes=[
            pltpu.VMEM((_C, _D), jnp.float32),
            pltpu.VMEM((_C, 1), jnp.float32),
        ],
    )(features, labels_row, fs_rows, sums, counts)

    return out[0, :4]
